# baseline (speedup 1.0000x reference)
import functools

import jax
import jax.numpy as jnp
from jax import lax
from jax.experimental import pallas as pl
from jax.experimental.pallas import tpu as pltpu
import jax.experimental.pallas.tpu_sc as plsc

N = 10000
D = 128
E = 320000

NC = 2
NS = 16
NW = NC * NS

CH = 128
ROWS_W = 80
SG = 40
EPAD = NW * ROWS_W * CH
NACC = 10112
STRIPE = NACC // NS


def _sc_degree_body(dstb, zeros, out, dst_v, ones_v, vbuf, obuf, acc):
    c = lax.axis_index("c")
    s = lax.axis_index("s")
    w = s * NC + c
    pltpu.sync_copy(
        zeros.at[pl.ds(s * STRIPE, STRIPE)], acc.at[pl.ds(s * STRIPE, STRIPE)]
    )

    one_hot = jnp.where(lax.iota(jnp.int32, 16) == 0, 1.0, 0.0).astype(jnp.float32)
    zero_vec = jnp.zeros((16,), jnp.float32)

    def fill_ones(k, carry):
        ones_v[k, pl.ds(0, 16)] = one_hot
        for j in range(1, D // 16):
            ones_v[k, pl.ds(16 * j, 16)] = zero_vec
        return carry

    lax.fori_loop(0, CH, fill_ones, 0)
    plsc.subcore_barrier()

    def outer(g, carry):
        pltpu.sync_copy(dstb.at[pl.ds(w * ROWS_W + g * 16, 16)], dst_v)

        def body(t, carry2):
            pltpu.sync_copy(ones_v, acc.at[dst_v.at[t]], add=True)
            return carry2

        lax.fori_loop(0, 16, body, 0)
        return carry

    lax.fori_loop(0, ROWS_W // 16, outer, 0)
    plsc.subcore_barrier()

    for b in range(5):
        sz = 128 if b < 4 else STRIPE - 4 * 128
        pltpu.sync_copy(
            acc.at[pl.ds(s * STRIPE + b * 128, sz)], vbuf.at[pl.ds(0, sz)]
        )

        def pack(q, carry):
            for j in range(8):
                obuf[b * 16 + q, pl.ds(16 * j, 16)] = vbuf[8 * q + j, pl.ds(0, 16)]
            return carry

        lax.fori_loop(0, sz // 8, pack, 0)

    pltpu.sync_copy(obuf, out.at[pl.ds((c * NS + s) * 80, 80)])


def _sc_scatter_body(
    m, srcb, dstb, zeros, out, src_v, dst_v, rows0, rows1, acc, sem0, sem1, sems0, sems1
):
    c = lax.axis_index("c")
    s = lax.axis_index("s")
    w = s * NC + c
    base = w * ROWS_W
    pltpu.sync_copy(
        zeros.at[pl.ds(s * STRIPE, STRIPE)], acc.at[pl.ds(s * STRIPE, STRIPE)]
    )
    plsc.subcore_barrier()

    def outer(g, carry):
        pltpu.sync_copy(srcb.at[pl.ds(base + g * SG, SG)], src_v)
        pltpu.sync_copy(dstb.at[pl.ds(base + g * SG, SG)], dst_v)
        pltpu.async_copy(m.at[src_v.at[0]], rows0, sem0)
        pltpu.async_copy(m.at[src_v.at[1]], rows1, sem1)

        def inner(i, carry2):
            t0 = 2 * i
            t1 = t0 + 1
            pltpu.make_async_copy(m.at[src_v.at[t0]], rows0, sem0).wait()
            h0 = pltpu.async_copy(rows0, acc.at[dst_v.at[t0]], sems0, add=True)
            pltpu.make_async_copy(m.at[src_v.at[t1]], rows1, sem1).wait()
            h1 = pltpu.async_copy(rows1, acc.at[dst_v.at[t1]], sems1, add=True)

            @pl.when(i < SG // 2 - 1)
            def _():
                h0.wait()
                pltpu.async_copy(m.at[src_v.at[t0 + 2]], rows0, sem0)
                h1.wait()
                pltpu.async_copy(m.at[src_v.at[t1 + 2]], rows1, sem1)

            @pl.when(i == SG // 2 - 1)
            def _():
                h0.wait()
                h1.wait()

            return carry2

        lax.fori_loop(0, SG // 2, inner, 0)
        return carry

    lax.fori_loop(0, ROWS_W // SG, outer, 0)
    plsc.subcore_barrier()
    pltpu.sync_copy(
        acc.at[pl.ds(s * STRIPE, STRIPE)],
        out.at[pl.ds(c * NACC + s * STRIPE, STRIPE)],
    )


@functools.lru_cache(maxsize=1)
def _sc_kernels():
    mesh = plsc.VectorSubcoreMesh(
        core_axis_name="c", subcore_axis_name="s", num_cores=NC, num_subcores=NS
    )
    degree = pl.kernel(
        _sc_degree_body,
        out_type=jax.ShapeDtypeStruct((NC * NS * 80, D), jnp.float32),
        mesh=mesh,
        scratch_types=[
            pltpu.VMEM((16, CH), jnp.int32),
            pltpu.VMEM((CH, D), jnp.float32),
            pltpu.VMEM((128, D), jnp.float32),
            pltpu.VMEM((80, D), jnp.float32),
            pltpu.VMEM_SHARED((NACC, D), jnp.float32),
        ],
    )
    scatter = pl.kernel(
        _sc_scatter_body,
        out_type=jax.ShapeDtypeStruct((NC * NACC, D), jnp.float32),
        mesh=mesh,
        scratch_types=[
            pltpu.VMEM((SG, CH), jnp.int32),
            pltpu.VMEM((SG, CH), jnp.int32),
            pltpu.VMEM((CH, D), jnp.float32),
            pltpu.VMEM((CH, D), jnp.float32),
            pltpu.VMEM_SHARED((NACC, D), jnp.float32),
            pltpu.SemaphoreType.DMA,
            pltpu.SemaphoreType.DMA,
            pltpu.SemaphoreType.DMA,
            pltpu.SemaphoreType.DMA,
        ],
    )
    return degree, scatter


_GRID = NACC // 128


def _dis_block(degr_ref):
    return lax.rsqrt(degr_ref[...])


def _tc_a_body(x_ref, w_ref, degp_ref, h_ref, m_ref):
    dis = _dis_block(degp_ref)
    h = lax.dot_general(
        x_ref[...], w_ref[...], (((1,), (1,)), ((), ())),
        preferred_element_type=jnp.float32,
    )
    h_ref[...] = h
    m_ref[...] = h * dis


def _tc_a(x_pad, W1, degp):
    return pl.pallas_call(
        _tc_a_body,
        grid=(_GRID,),
        in_specs=[
            pl.BlockSpec((128, D), lambda i: (i, 0)),
            pl.BlockSpec((D, D), lambda i: (0, 0)),
            pl.BlockSpec((128, D), lambda i: (i, 0)),
        ],
        out_specs=[
            pl.BlockSpec((128, D), lambda i: (i, 0)),
            pl.BlockSpec((128, D), lambda i: (i, 0)),
        ],
        out_shape=[
            jax.ShapeDtypeStruct((NACC, D), jnp.float32),
            jax.ShapeDtypeStruct((NACC, D), jnp.float32),
        ],
    )(x_pad, W1, degp)


def _tc_b_body(acc_ref, h1_ref, degp_ref, b1_ref, w_ref, h2_ref, m2_ref):
    i = pl.program_id(0)
    dis = _dis_block(degp_ref)
    a = acc_ref[0] + acc_ref[1]
    z = jnp.maximum(dis * a + dis * dis * h1_ref[...] + b1_ref[...], 0.0)
    rid = lax.broadcasted_iota(jnp.int32, (128, 1), 0) + i * 128
    z = jnp.where(rid < N, z, 0.0)
    h2 = lax.dot_general(
        z, w_ref[...], (((1,), (1,)), ((), ())),
        preferred_element_type=jnp.float32,
    )
    h2_ref[...] = h2
    m2_ref[...] = h2 * dis


def _tc_b(acc1, h1, degp, b1, W2):
    return pl.pallas_call(
        _tc_b_body,
        grid=(_GRID,),
        in_specs=[
            pl.BlockSpec((NC, 128, D), lambda i: (0, i, 0)),
            pl.BlockSpec((128, D), lambda i: (i, 0)),
            pl.BlockSpec((128, D), lambda i: (i, 0)),
            pl.BlockSpec((1, D), lambda i: (0, 0)),
            pl.BlockSpec((D, D), lambda i: (0, 0)),
        ],
        out_specs=[
            pl.BlockSpec((128, D), lambda i: (i, 0)),
            pl.BlockSpec((128, D), lambda i: (i, 0)),
        ],
        out_shape=[
            jax.ShapeDtypeStruct((NACC, D), jnp.float32),
            jax.ShapeDtypeStruct((NACC, D), jnp.float32),
        ],
    )(acc1, h1, degp, b1, W2)


def _tc_c_body(acc_ref, h2_ref, degp_ref, b2_ref, out_ref):
    dis = _dis_block(degp_ref)
    a = acc_ref[0] + acc_ref[1]
    out_ref[...] = dis * a + dis * dis * h2_ref[...] + b2_ref[...]


def _tc_c(acc2, h2, degp, b2):
    return pl.pallas_call(
        _tc_c_body,
        grid=(_GRID,),
        in_specs=[
            pl.BlockSpec((NC, 128, D), lambda i: (0, i, 0)),
            pl.BlockSpec((128, D), lambda i: (i, 0)),
            pl.BlockSpec((128, D), lambda i: (i, 0)),
            pl.BlockSpec((1, D), lambda i: (0, 0)),
        ],
        out_specs=pl.BlockSpec((128, D), lambda i: (i, 0)),
        out_shape=jax.ShapeDtypeStruct((NACC, D), jnp.float32),
    )(acc2, h2, degp, b2)


def kernel(x, edge_index, W1, b1, W2, b2):
    src = edge_index[0].astype(jnp.int32)
    dst = edge_index[1].astype(jnp.int32)

    npad = EPAD - E
    pad_idx = (N + jnp.arange(npad, dtype=jnp.int32) % (NACC - N)).astype(jnp.int32)
    srcb = jnp.concatenate([src, pad_idx]).reshape(EPAD // CH, CH)
    dstb = jnp.concatenate([dst, pad_idx]).reshape(EPAD // CH, CH)

    x_pad = jnp.concatenate(
        [x, jnp.zeros((NACC - N, D), jnp.float32)], axis=0
    )
    zeros = jnp.zeros((NACC, D), jnp.float32)
    b1r = b1.reshape(1, D)
    b2r = b2.reshape(1, D)

    sc_degree, sc_scatter = _sc_kernels()
    degp = sc_degree(dstb, zeros)
    deg = (
        degp.reshape(NC, NS, 80, D)[:, :, : STRIPE // 8, :]
        .reshape(NC, NS, STRIPE, 16)[:, :, :, 0]
        .reshape(NC, NACC)
        .sum(axis=0)
        + 1.0
    )
    degr = jnp.broadcast_to(deg[:, None], (NACC, D))
    h1, m1 = _tc_a(x_pad, W1, degr)
    acc1 = sc_scatter(m1, srcb, dstb, zeros).reshape(NC, NACC, D)
    h2, m2 = _tc_b(acc1, h1, degr, b1r, W2)
    acc2 = sc_scatter(m2, srcb, dstb, zeros).reshape(NC, NACC, D)
    out = _tc_c(acc2, h2, degr, b2r)
    return out[:N]


# speedup vs baseline: 1.1437x; 1.1437x over previous
import functools

import jax
import jax.numpy as jnp
from jax import lax
from jax.experimental import pallas as pl
from jax.experimental.pallas import tpu as pltpu
import jax.experimental.pallas.tpu_sc as plsc

N = 10000
D = 128
E = 320000

NC = 2
NS = 16
NW = NC * NS

CH = 128
ROWS_W = 80
SG = 40
EPAD = NW * ROWS_W * CH
NACC = 10112
STRIPE = NACC // NS


def _sc_degree_body(dstb, zeros, out, dst_v, ones_v, vbuf, obuf, acc):
    c = lax.axis_index("c")
    s = lax.axis_index("s")
    w = s * NC + c
    pltpu.sync_copy(
        zeros.at[pl.ds(s * STRIPE, STRIPE)], acc.at[pl.ds(s * STRIPE, STRIPE)]
    )

    one_hot = jnp.where(lax.iota(jnp.int32, 16) == 0, 1.0, 0.0).astype(jnp.float32)
    zero_vec = jnp.zeros((16,), jnp.float32)

    def fill_ones(k, carry):
        ones_v[k, pl.ds(0, 16)] = one_hot
        for j in range(1, D // 16):
            ones_v[k, pl.ds(16 * j, 16)] = zero_vec
        return carry

    lax.fori_loop(0, CH, fill_ones, 0)
    plsc.subcore_barrier()

    def outer(g, carry):
        pltpu.sync_copy(dstb.at[pl.ds(w * ROWS_W + g * 16, 16)], dst_v)

        def body(t, carry2):
            pltpu.sync_copy(ones_v, acc.at[dst_v.at[t]], add=True)
            return carry2

        lax.fori_loop(0, 16, body, 0)
        return carry

    lax.fori_loop(0, ROWS_W // 16, outer, 0)
    plsc.subcore_barrier()

    for b in range(5):
        sz = 128 if b < 4 else STRIPE - 4 * 128
        pltpu.sync_copy(
            acc.at[pl.ds(s * STRIPE + b * 128, sz)], vbuf.at[pl.ds(0, sz)]
        )

        def pack(q, carry):
            for j in range(8):
                obuf[b * 16 + q, pl.ds(16 * j, 16)] = vbuf[8 * q + j, pl.ds(0, 16)]
            return carry

        lax.fori_loop(0, sz // 8, pack, 0)

    pltpu.sync_copy(obuf, out.at[pl.ds((c * NS + s) * 80, 80)])


def _sc_scatter_body(
    m, srcb, dstb, zeros, out, src_v, dst_v, rows0, rows1, acc, sem0, sem1, sems0, sems1
):
    c = lax.axis_index("c")
    s = lax.axis_index("s")
    w = s * NC + c
    base = w * ROWS_W
    pltpu.sync_copy(
        zeros.at[pl.ds(s * STRIPE, STRIPE)], acc.at[pl.ds(s * STRIPE, STRIPE)]
    )
    plsc.subcore_barrier()

    def outer(g, carry):
        pltpu.sync_copy(srcb.at[pl.ds(base + g * SG, SG)], src_v)
        pltpu.sync_copy(dstb.at[pl.ds(base + g * SG, SG)], dst_v)
        pltpu.async_copy(m.at[src_v.at[0]], rows0, sem0)

        def inner(i, carry2):
            t0 = 2 * i
            t1 = t0 + 1
            pltpu.async_copy(m.at[src_v.at[t1]], rows1, sem1)
            pltpu.make_async_copy(m.at[src_v.at[t0]], rows0, sem0).wait()
            pltpu.sync_copy(rows0, acc.at[dst_v.at[t0]], add=True)

            @pl.when(i < SG // 2 - 1)
            def _():
                pltpu.async_copy(m.at[src_v.at[t1 + 1]], rows0, sem0)

            pltpu.make_async_copy(m.at[src_v.at[t1]], rows1, sem1).wait()
            pltpu.sync_copy(rows1, acc.at[dst_v.at[t1]], add=True)
            return carry2

        lax.fori_loop(0, SG // 2, inner, 0)
        return carry

    lax.fori_loop(0, ROWS_W // SG, outer, 0)
    plsc.subcore_barrier()
    pltpu.sync_copy(
        acc.at[pl.ds(s * STRIPE, STRIPE)],
        out.at[pl.ds(c * NACC + s * STRIPE, STRIPE)],
    )


@functools.lru_cache(maxsize=1)
def _sc_kernels():
    mesh = plsc.VectorSubcoreMesh(
        core_axis_name="c", subcore_axis_name="s", num_cores=NC, num_subcores=NS
    )
    degree = pl.kernel(
        _sc_degree_body,
        out_type=jax.ShapeDtypeStruct((NC * NS * 80, D), jnp.float32),
        mesh=mesh,
        scratch_types=[
            pltpu.VMEM((16, CH), jnp.int32),
            pltpu.VMEM((CH, D), jnp.float32),
            pltpu.VMEM((128, D), jnp.float32),
            pltpu.VMEM((80, D), jnp.float32),
            pltpu.VMEM_SHARED((NACC, D), jnp.float32),
        ],
    )
    scatter = pl.kernel(
        _sc_scatter_body,
        out_type=jax.ShapeDtypeStruct((NC * NACC, D), jnp.float32),
        mesh=mesh,
        scratch_types=[
            pltpu.VMEM((SG, CH), jnp.int32),
            pltpu.VMEM((SG, CH), jnp.int32),
            pltpu.VMEM((CH, D), jnp.float32),
            pltpu.VMEM((CH, D), jnp.float32),
            pltpu.VMEM_SHARED((NACC, D), jnp.float32),
            pltpu.SemaphoreType.DMA,
            pltpu.SemaphoreType.DMA,
            pltpu.SemaphoreType.DMA,
            pltpu.SemaphoreType.DMA,
        ],
    )
    return degree, scatter


_GRID = NACC // 128


def _dis_block(degr_ref):
    return lax.rsqrt(degr_ref[...])


def _tc_a_body(x_ref, w_ref, degp_ref, h_ref, m_ref):
    dis = _dis_block(degp_ref)
    h = lax.dot_general(
        x_ref[...], w_ref[...], (((1,), (1,)), ((), ())),
        preferred_element_type=jnp.float32,
    )
    h_ref[...] = h
    m_ref[...] = h * dis


def _tc_a(x_pad, W1, degp):
    return pl.pallas_call(
        _tc_a_body,
        grid=(_GRID,),
        in_specs=[
            pl.BlockSpec((128, D), lambda i: (i, 0)),
            pl.BlockSpec((D, D), lambda i: (0, 0)),
            pl.BlockSpec((128, D), lambda i: (i, 0)),
        ],
        out_specs=[
            pl.BlockSpec((128, D), lambda i: (i, 0)),
            pl.BlockSpec((128, D), lambda i: (i, 0)),
        ],
        out_shape=[
            jax.ShapeDtypeStruct((NACC, D), jnp.float32),
            jax.ShapeDtypeStruct((NACC, D), jnp.float32),
        ],
    )(x_pad, W1, degp)


def _tc_b_body(acc_ref, h1_ref, degp_ref, b1_ref, w_ref, h2_ref, m2_ref):
    i = pl.program_id(0)
    dis = _dis_block(degp_ref)
    a = acc_ref[0] + acc_ref[1]
    z = jnp.maximum(dis * a + dis * dis * h1_ref[...] + b1_ref[...], 0.0)
    rid = lax.broadcasted_iota(jnp.int32, (128, 1), 0) + i * 128
    z = jnp.where(rid < N, z, 0.0)
    h2 = lax.dot_general(
        z, w_ref[...], (((1,), (1,)), ((), ())),
        preferred_element_type=jnp.float32,
    )
    h2_ref[...] = h2
    m2_ref[...] = h2 * dis


def _tc_b(acc1, h1, degp, b1, W2):
    return pl.pallas_call(
        _tc_b_body,
        grid=(_GRID,),
        in_specs=[
            pl.BlockSpec((NC, 128, D), lambda i: (0, i, 0)),
            pl.BlockSpec((128, D), lambda i: (i, 0)),
            pl.BlockSpec((128, D), lambda i: (i, 0)),
            pl.BlockSpec((1, D), lambda i: (0, 0)),
            pl.BlockSpec((D, D), lambda i: (0, 0)),
        ],
        out_specs=[
            pl.BlockSpec((128, D), lambda i: (i, 0)),
            pl.BlockSpec((128, D), lambda i: (i, 0)),
        ],
        out_shape=[
            jax.ShapeDtypeStruct((NACC, D), jnp.float32),
            jax.ShapeDtypeStruct((NACC, D), jnp.float32),
        ],
    )(acc1, h1, degp, b1, W2)


def _tc_c_body(acc_ref, h2_ref, degp_ref, b2_ref, out_ref):
    dis = _dis_block(degp_ref)
    a = acc_ref[0] + acc_ref[1]
    out_ref[...] = dis * a + dis * dis * h2_ref[...] + b2_ref[...]


def _tc_c(acc2, h2, degp, b2):
    return pl.pallas_call(
        _tc_c_body,
        grid=(_GRID,),
        in_specs=[
            pl.BlockSpec((NC, 128, D), lambda i: (0, i, 0)),
            pl.BlockSpec((128, D), lambda i: (i, 0)),
            pl.BlockSpec((128, D), lambda i: (i, 0)),
            pl.BlockSpec((1, D), lambda i: (0, 0)),
        ],
        out_specs=pl.BlockSpec((128, D), lambda i: (i, 0)),
        out_shape=jax.ShapeDtypeStruct((NACC, D), jnp.float32),
    )(acc2, h2, degp, b2)


def kernel(x, edge_index, W1, b1, W2, b2):
    src = edge_index[0].astype(jnp.int32)
    dst = edge_index[1].astype(jnp.int32)

    npad = EPAD - E
    pad_idx = (N + jnp.arange(npad, dtype=jnp.int32) % (NACC - N)).astype(jnp.int32)
    srcb = jnp.concatenate([src, pad_idx]).reshape(EPAD // CH, CH)
    dstb = jnp.concatenate([dst, pad_idx]).reshape(EPAD // CH, CH)

    x_pad = jnp.concatenate(
        [x, jnp.zeros((NACC - N, D), jnp.float32)], axis=0
    )
    zeros = jnp.zeros((NACC, D), jnp.float32)
    b1r = b1.reshape(1, D)
    b2r = b2.reshape(1, D)

    sc_degree, sc_scatter = _sc_kernels()
    degp = sc_degree(dstb, zeros)
    deg = (
        degp.reshape(NC, NS, 80, D)[:, :, : STRIPE // 8, :]
        .reshape(NC, NS, STRIPE, 16)[:, :, :, 0]
        .reshape(NC, NACC)
        .sum(axis=0)
        + 1.0
    )
    degr = jnp.broadcast_to(deg[:, None], (NACC, D))
    h1, m1 = _tc_a(x_pad, W1, degr)
    acc1 = sc_scatter(m1, srcb, dstb, zeros).reshape(NC, NACC, D)
    h2, m2 = _tc_b(acc1, h1, degr, b1r, W2)
    acc2 = sc_scatter(m2, srcb, dstb, zeros).reshape(NC, NACC, D)
    out = _tc_c(acc2, h2, degr, b2r)
    return out[:N]


# speedup vs baseline: 1.1787x; 1.0306x over previous
import functools

import jax
import jax.numpy as jnp
from jax import lax
from jax.experimental import pallas as pl
from jax.experimental.pallas import tpu as pltpu
import jax.experimental.pallas.tpu_sc as plsc

N = 10000
D = 128
E = 320000

NC = 2
NS = 16
NW = NC * NS

CH = 128
ROWS_W = 80
SG = 40
EPAD = NW * ROWS_W * CH
NACC = 10112
STRIPE = NACC // NS


def _sc_degree_body(dstb, zeros, out, dst_v, ones_v, acc):
    c = lax.axis_index("c")
    s = lax.axis_index("s")
    w = s * NC + c
    pltpu.sync_copy(
        zeros.at[pl.ds(s * STRIPE, STRIPE)], acc.at[pl.ds(s * STRIPE, STRIPE)]
    )

    ones16 = jnp.ones((16,), jnp.float32)

    def fill_ones(k, carry):
        for j in range(D // 16):
            ones_v[k, pl.ds(16 * j, 16)] = ones16
        return carry

    lax.fori_loop(0, CH, fill_ones, 0)
    plsc.subcore_barrier()

    def outer(g, carry):
        pltpu.sync_copy(dstb.at[pl.ds(w * ROWS_W + g * 16, 16)], dst_v)

        def body(t, carry2):
            pltpu.sync_copy(ones_v, acc.at[dst_v.at[t]], add=True)
            return carry2

        lax.fori_loop(0, 16, body, 0)
        return carry

    lax.fori_loop(0, ROWS_W // 16, outer, 0)
    plsc.subcore_barrier()
    pltpu.sync_copy(
        acc.at[pl.ds(s * STRIPE, STRIPE)],
        out.at[pl.ds(c * NACC + s * STRIPE, STRIPE)],
    )


def _sc_scatter_body(
    m, srcb, dstb, zeros, out, src_v, dst_v, rows0, rows1, acc, sem0, sem1, sems0, sems1
):
    c = lax.axis_index("c")
    s = lax.axis_index("s")
    w = s * NC + c
    base = w * ROWS_W
    pltpu.sync_copy(
        zeros.at[pl.ds(s * STRIPE, STRIPE)], acc.at[pl.ds(s * STRIPE, STRIPE)]
    )
    plsc.subcore_barrier()

    def outer(g, carry):
        pltpu.sync_copy(srcb.at[pl.ds(base + g * SG, SG)], src_v)
        pltpu.sync_copy(dstb.at[pl.ds(base + g * SG, SG)], dst_v)
        pltpu.async_copy(m.at[src_v.at[0]], rows0, sem0)

        def inner(i, carry2):
            t0 = 2 * i
            t1 = t0 + 1
            pltpu.async_copy(m.at[src_v.at[t1]], rows1, sem1)
            pltpu.make_async_copy(m.at[src_v.at[t0]], rows0, sem0).wait()
            pltpu.sync_copy(rows0, acc.at[dst_v.at[t0]], add=True)

            @pl.when(i < SG // 2 - 1)
            def _():
                pltpu.async_copy(m.at[src_v.at[t1 + 1]], rows0, sem0)

            pltpu.make_async_copy(m.at[src_v.at[t1]], rows1, sem1).wait()
            pltpu.sync_copy(rows1, acc.at[dst_v.at[t1]], add=True)
            return carry2

        lax.fori_loop(0, SG // 2, inner, 0)
        return carry

    lax.fori_loop(0, ROWS_W // SG, outer, 0)
    plsc.subcore_barrier()
    pltpu.sync_copy(
        acc.at[pl.ds(s * STRIPE, STRIPE)],
        out.at[pl.ds(c * NACC + s * STRIPE, STRIPE)],
    )


@functools.lru_cache(maxsize=1)
def _sc_kernels():
    mesh = plsc.VectorSubcoreMesh(
        core_axis_name="c", subcore_axis_name="s", num_cores=NC, num_subcores=NS
    )
    degree = pl.kernel(
        _sc_degree_body,
        out_type=jax.ShapeDtypeStruct((NC * NACC, D), jnp.float32),
        mesh=mesh,
        scratch_types=[
            pltpu.VMEM((16, CH), jnp.int32),
            pltpu.VMEM((CH, D), jnp.float32),
            pltpu.VMEM_SHARED((NACC, D), jnp.float32),
        ],
    )
    scatter = pl.kernel(
        _sc_scatter_body,
        out_type=jax.ShapeDtypeStruct((NC * NACC, D), jnp.float32),
        mesh=mesh,
        scratch_types=[
            pltpu.VMEM((SG, CH), jnp.int32),
            pltpu.VMEM((SG, CH), jnp.int32),
            pltpu.VMEM((CH, D), jnp.float32),
            pltpu.VMEM((CH, D), jnp.float32),
            pltpu.VMEM_SHARED((NACC, D), jnp.float32),
            pltpu.SemaphoreType.DMA,
            pltpu.SemaphoreType.DMA,
            pltpu.SemaphoreType.DMA,
            pltpu.SemaphoreType.DMA,
        ],
    )
    return degree, scatter


_GRID = NACC // 128


def _dis_block(degp_ref):
    return lax.rsqrt(degp_ref[0] + degp_ref[1] + 1.0)


def _tc_a_body(x_ref, w_ref, degp_ref, h_ref, m_ref):
    dis = _dis_block(degp_ref)
    h = lax.dot_general(
        x_ref[...], w_ref[...], (((1,), (1,)), ((), ())),
        preferred_element_type=jnp.float32,
    )
    h_ref[...] = h
    m_ref[...] = h * dis


def _tc_a(x_pad, W1, degp):
    return pl.pallas_call(
        _tc_a_body,
        grid=(_GRID,),
        in_specs=[
            pl.BlockSpec((128, D), lambda i: (i, 0)),
            pl.BlockSpec((D, D), lambda i: (0, 0)),
            pl.BlockSpec((NC, 128, D), lambda i: (0, i, 0)),
        ],
        out_specs=[
            pl.BlockSpec((128, D), lambda i: (i, 0)),
            pl.BlockSpec((128, D), lambda i: (i, 0)),
        ],
        out_shape=[
            jax.ShapeDtypeStruct((NACC, D), jnp.float32),
            jax.ShapeDtypeStruct((NACC, D), jnp.float32),
        ],
    )(x_pad, W1, degp)


def _tc_b_body(acc_ref, h1_ref, degp_ref, b1_ref, w_ref, h2_ref, m2_ref):
    i = pl.program_id(0)
    dis = _dis_block(degp_ref)
    a = acc_ref[0] + acc_ref[1]
    z = jnp.maximum(dis * a + dis * dis * h1_ref[...] + b1_ref[...], 0.0)
    rid = lax.broadcasted_iota(jnp.int32, (128, 1), 0) + i * 128
    z = jnp.where(rid < N, z, 0.0)
    h2 = lax.dot_general(
        z, w_ref[...], (((1,), (1,)), ((), ())),
        preferred_element_type=jnp.float32,
    )
    h2_ref[...] = h2
    m2_ref[...] = h2 * dis


def _tc_b(acc1, h1, degp, b1, W2):
    return pl.pallas_call(
        _tc_b_body,
        grid=(_GRID,),
        in_specs=[
            pl.BlockSpec((NC, 128, D), lambda i: (0, i, 0)),
            pl.BlockSpec((128, D), lambda i: (i, 0)),
            pl.BlockSpec((NC, 128, D), lambda i: (0, i, 0)),
            pl.BlockSpec((1, D), lambda i: (0, 0)),
            pl.BlockSpec((D, D), lambda i: (0, 0)),
        ],
        out_specs=[
            pl.BlockSpec((128, D), lambda i: (i, 0)),
            pl.BlockSpec((128, D), lambda i: (i, 0)),
        ],
        out_shape=[
            jax.ShapeDtypeStruct((NACC, D), jnp.float32),
            jax.ShapeDtypeStruct((NACC, D), jnp.float32),
        ],
    )(acc1, h1, degp, b1, W2)


def _tc_c_body(acc_ref, h2_ref, degp_ref, b2_ref, out_ref):
    dis = _dis_block(degp_ref)
    a = acc_ref[0] + acc_ref[1]
    out_ref[...] = dis * a + dis * dis * h2_ref[...] + b2_ref[...]


def _tc_c(acc2, h2, degp, b2):
    return pl.pallas_call(
        _tc_c_body,
        grid=(_GRID,),
        in_specs=[
            pl.BlockSpec((NC, 128, D), lambda i: (0, i, 0)),
            pl.BlockSpec((128, D), lambda i: (i, 0)),
            pl.BlockSpec((NC, 128, D), lambda i: (0, i, 0)),
            pl.BlockSpec((1, D), lambda i: (0, 0)),
        ],
        out_specs=pl.BlockSpec((128, D), lambda i: (i, 0)),
        out_shape=jax.ShapeDtypeStruct((NACC, D), jnp.float32),
    )(acc2, h2, degp, b2)


def kernel(x, edge_index, W1, b1, W2, b2):
    src = edge_index[0].astype(jnp.int32)
    dst = edge_index[1].astype(jnp.int32)

    npad = EPAD - E
    pad_idx = (N + jnp.arange(npad, dtype=jnp.int32) % (NACC - N)).astype(jnp.int32)
    srcb = jnp.concatenate([src, pad_idx]).reshape(EPAD // CH, CH)
    dstb = jnp.concatenate([dst, pad_idx]).reshape(EPAD // CH, CH)

    x_pad = jnp.concatenate(
        [x, jnp.zeros((NACC - N, D), jnp.float32)], axis=0
    )
    zeros = jnp.zeros((NACC, D), jnp.float32)
    b1r = b1.reshape(1, D)
    b2r = b2.reshape(1, D)

    sc_degree, sc_scatter = _sc_kernels()
    degp = sc_degree(dstb, zeros).reshape(NC, NACC, D)
    h1, m1 = _tc_a(x_pad, W1, degp)
    acc1 = sc_scatter(m1, srcb, dstb, zeros).reshape(NC, NACC, D)
    h2, m2 = _tc_b(acc1, h1, degp, b1r, W2)
    acc2 = sc_scatter(m2, srcb, dstb, zeros).reshape(NC, NACC, D)
    out = _tc_c(acc2, h2, degp, b2r)
    return out[:N]
